# baseline (speedup 1.0000x reference)
import functools

import jax
import jax.numpy as jnp
from jax import lax
from jax.experimental import pallas as pl
from jax.experimental.pallas import tpu as pltpu
from jax.experimental.pallas import tpu_sc as plsc

N = 10000
E = 320000
D = 128

NC = 2
NS = 16
NW = NC * NS
EPW = E // NW
CHUNK = 80
NCHUNK = EPW // CHUNK
RPT = 624
REM0 = NS * RPT
REM = N - REM0

_MESH = plsc.VectorSubcoreMesh(core_axis_name="c", subcore_axis_name="s")


def _make_sc_aggregate(width):
    @functools.partial(
        pl.kernel,
        out_type=jax.ShapeDtypeStruct((NC, N, width), jnp.float32),
        mesh=_MESH,
        scratch_types=[
            pltpu.VMEM((CHUNK,), jnp.int32),
            pltpu.VMEM((CHUNK,), jnp.int32),
            pltpu.VMEM((CHUNK, width), jnp.float32),
            pltpu.VMEM_SHARED((N, width), jnp.float32),
            pltpu.SemaphoreType.DMA,
        ],
    )
    def agg(src_hbm, dst_hbm, tbl_hbm, zeros_hbm, out_hbm,
            sidx_v, didx_v, rows_v, acc_sp, sem):
        c = lax.axis_index("c")
        s = lax.axis_index("s")
        wid = c * NS + s
        row0 = s * RPT
        pltpu.sync_copy(zeros_hbm.at[pl.ds(row0, RPT)],
                        acc_sp.at[pl.ds(row0, RPT)])

        @pl.when(s == NS - 1)
        def _():
            pltpu.sync_copy(zeros_hbm.at[pl.ds(REM0, REM)],
                            acc_sp.at[pl.ds(REM0, REM)])

        plsc.subcore_barrier()
        base = wid * EPW

        def body(i, carry):
            e0 = base + i * CHUNK
            pltpu.sync_copy(src_hbm.at[pl.ds(e0, CHUNK)], sidx_v)
            pltpu.sync_copy(dst_hbm.at[pl.ds(e0, CHUNK)], didx_v)
            pltpu.async_copy(tbl_hbm.at[sidx_v], rows_v, sem).wait()
            pltpu.sync_copy(rows_v, acc_sp.at[didx_v], add=True)
            return carry

        lax.fori_loop(0, NCHUNK, body, 0)
        plsc.subcore_barrier()
        pltpu.sync_copy(acc_sp.at[pl.ds(row0, RPT)],
                        out_hbm.at[c, pl.ds(row0, RPT)])

        @pl.when(s == NS - 1)
        def _():
            pltpu.sync_copy(acc_sp.at[pl.ds(REM0, REM)],
                            out_hbm.at[c, pl.ds(REM0, REM)])

    return agg


_sc_aggregate = _make_sc_aggregate(D)
_sc_degree = _make_sc_aggregate(D)


_B = 1000
_GRID = N // _B


def _tc_first_body(x_ref, w_ref, d0_ref, d1_ref, hp_ref, dinv_ref):
    deg = d0_ref[:, :1] + d1_ref[:, :1] + 1.0
    dinv = lax.rsqrt(deg)
    h = jnp.dot(x_ref[...], w_ref[...], preferred_element_type=jnp.float32)
    hp_ref[...] = dinv * h
    dinv_ref[...] = dinv


def _tc_mid_body(s0_ref, s1_ref, hp_ref, dinv_ref, b_ref, w_ref, out_ref):
    dinv = dinv_ref[...]
    a = jax.nn.relu(dinv * (s0_ref[...] + s1_ref[...] + hp_ref[...]) + b_ref[...])
    out_ref[...] = dinv * jnp.dot(a, w_ref[...], preferred_element_type=jnp.float32)


def _tc_final_body(s0_ref, s1_ref, hp_ref, dinv_ref, b_ref, out_ref):
    out_ref[...] = (dinv_ref[...] * (s0_ref[...] + s1_ref[...] + hp_ref[...])
                    + b_ref[...])


_ROWS = pl.BlockSpec((_B, D), lambda i: (i, 0))
_COL1 = pl.BlockSpec((_B, 1), lambda i: (i, 0))

_WMAT = pl.BlockSpec((D, D), lambda i: (0, 0))
_BVEC = pl.BlockSpec((1, D), lambda i: (0, 0))

_tc_first = pl.pallas_call(
    _tc_first_body,
    grid=(_GRID,),
    in_specs=[_ROWS, _WMAT, _ROWS, _ROWS],
    out_specs=[_ROWS, _COL1],
    out_shape=[
        jax.ShapeDtypeStruct((N, D), jnp.float32),
        jax.ShapeDtypeStruct((N, 1), jnp.float32),
    ],
)

_tc_mid = pl.pallas_call(
    _tc_mid_body,
    grid=(_GRID,),
    in_specs=[_ROWS, _ROWS, _ROWS, _COL1, _BVEC, _WMAT],
    out_specs=_ROWS,
    out_shape=jax.ShapeDtypeStruct((N, D), jnp.float32),
)

_tc_final = pl.pallas_call(
    _tc_final_body,
    grid=(_GRID,),
    in_specs=[_ROWS, _ROWS, _ROWS, _COL1, _BVEC],
    out_specs=_ROWS,
    out_shape=jax.ShapeDtypeStruct((N, D), jnp.float32),
)


def kernel(x, edge_index, W1, b1, W2, b2, W3, b3):
    src = edge_index[0]
    dst = edge_index[1]
    onesD = jnp.ones((N, D), jnp.float32)
    zerosD = jnp.zeros((N, D), jnp.float32)

    deg2 = _sc_degree(dst, dst, onesD, zerosD)
    hp, dinv = _tc_first(x, W1, deg2[0], deg2[1])

    S = _sc_aggregate(src, dst, hp, zerosD)
    hp = _tc_mid(S[0], S[1], hp, dinv, b1.reshape(1, D), W2)

    S = _sc_aggregate(src, dst, hp, zerosD)
    hp = _tc_mid(S[0], S[1], hp, dinv, b2.reshape(1, D), W3)

    S = _sc_aggregate(src, dst, hp, zerosD)
    return _tc_final(S[0], S[1], hp, dinv, b3.reshape(1, D))


# speedup vs baseline: 1.9909x; 1.9909x over previous
import functools

import jax
import jax.numpy as jnp
from jax import lax
from jax.experimental import pallas as pl
from jax.experimental.pallas import tpu as pltpu
from jax.experimental.pallas import tpu_sc as plsc

N = 10000
E = 320000
D = 128

NC = 2
NS = 16
NW = NC * NS
EPW = E // NW
CHUNK = 80
NCHUNK = EPW // CHUNK
RPT = 624
REM0 = NS * RPT
REM = N - REM0

_MESH = plsc.VectorSubcoreMesh(core_axis_name="c", subcore_axis_name="s")


NBUF = 4
NROUND = NCHUNK // NBUF
NTAIL = NCHUNK - NROUND * NBUF


def _make_sc_aggregate(width):
    @functools.partial(
        pl.kernel,
        out_type=jax.ShapeDtypeStruct((NC, N, width), jnp.float32),
        mesh=_MESH,
        scratch_types=[pltpu.VMEM_SHARED((N, width), jnp.float32)]
          + [pltpu.VMEM((CHUNK,), jnp.int32)] * (2 * NBUF)
          + [pltpu.VMEM((CHUNK, width), jnp.float32)] * NBUF
          + [pltpu.SemaphoreType.DMA] * (3 * NBUF),
    )
    def agg(src_hbm, dst_hbm, tbl_hbm, zeros_hbm, out_hbm, acc_sp, *rest):
        sidx = rest[:NBUF]
        didx = rest[NBUF:2 * NBUF]
        rows = rest[2 * NBUF:3 * NBUF]
        semi = rest[3 * NBUF:4 * NBUF]
        semj = rest[4 * NBUF:5 * NBUF]
        semg = rest[5 * NBUF:6 * NBUF]
        c = lax.axis_index("c")
        s = lax.axis_index("s")
        wid = c * NS + s
        row0 = s * RPT
        pltpu.sync_copy(zeros_hbm.at[pl.ds(row0, RPT)],
                        acc_sp.at[pl.ds(row0, RPT)])

        @pl.when(s == NS - 1)
        def _():
            pltpu.sync_copy(zeros_hbm.at[pl.ds(REM0, REM)],
                            acc_sp.at[pl.ds(REM0, REM)])

        base = wid * EPW

        def idx_start(j, b):
            e0 = base + j * CHUNK
            pltpu.async_copy(src_hbm.at[pl.ds(e0, CHUNK)], sidx[b], semi[b])
            pltpu.async_copy(dst_hbm.at[pl.ds(e0, CHUNK)], didx[b], semj[b])

        def idx_drain(b):
            pltpu.make_async_copy(src_hbm.at[pl.ds(0, CHUNK)], sidx[b],
                                  semi[b]).wait()
            pltpu.make_async_copy(dst_hbm.at[pl.ds(0, CHUNK)], didx[b],
                                  semj[b]).wait()

        plsc.subcore_barrier()
        for b in range(NBUF):
            idx_start(b, b)

        def body(i, carry):
            gats = []
            for b in range(NBUF):
                idx_drain(b)
                gats.append(
                    pltpu.async_copy(tbl_hbm.at[sidx[b]], rows[b], semg[b]))
            for b in range(NBUF):
                gats[b].wait()
                pltpu.sync_copy(rows[b], acc_sp.at[didx[b]], add=True)
                jn = (i + 1) * NBUF + b
                if b < NTAIL:
                    idx_start(jn, b)
                else:
                    @pl.when(i < NROUND - 1)
                    def _():
                        idx_start(jn, b)
            return carry

        lax.fori_loop(0, NROUND, body, 0)
        for b in range(NTAIL):
            idx_drain(b)
            pltpu.async_copy(tbl_hbm.at[sidx[b]], rows[b], semg[b]).wait()
            pltpu.sync_copy(rows[b], acc_sp.at[didx[b]], add=True)
        plsc.subcore_barrier()
        pltpu.sync_copy(acc_sp.at[pl.ds(row0, RPT)],
                        out_hbm.at[c, pl.ds(row0, RPT)])

        @pl.when(s == NS - 1)
        def _():
            pltpu.sync_copy(acc_sp.at[pl.ds(REM0, REM)],
                            out_hbm.at[c, pl.ds(REM0, REM)])

    return agg


_sc_aggregate = _make_sc_aggregate(D)


_B = 1000
_GRID = N // _B


def _tc_first_body(x_ref, w_ref, d0_ref, d1_ref, hp_ref, dinv_ref):
    deg = d0_ref[:, :1] + d1_ref[:, :1] + 1.0
    dinv = lax.rsqrt(deg)
    h = jnp.dot(x_ref[...], w_ref[...], preferred_element_type=jnp.float32)
    hp_ref[...] = dinv * h
    dinv_ref[...] = dinv


def _tc_mid_body(s0_ref, s1_ref, hp_ref, dinv_ref, b_ref, w_ref, out_ref):
    dinv = dinv_ref[...]
    a = jax.nn.relu(dinv * (s0_ref[...] + s1_ref[...] + hp_ref[...]) + b_ref[...])
    out_ref[...] = dinv * jnp.dot(a, w_ref[...], preferred_element_type=jnp.float32)


def _tc_final_body(s0_ref, s1_ref, hp_ref, dinv_ref, b_ref, out_ref):
    out_ref[...] = (dinv_ref[...] * (s0_ref[...] + s1_ref[...] + hp_ref[...])
                    + b_ref[...])


_ROWS = pl.BlockSpec((_B, D), lambda i: (i, 0))
_COL1 = pl.BlockSpec((_B, 1), lambda i: (i, 0))

_WMAT = pl.BlockSpec((D, D), lambda i: (0, 0))
_BVEC = pl.BlockSpec((1, D), lambda i: (0, 0))

_tc_first = pl.pallas_call(
    _tc_first_body,
    grid=(_GRID,),
    in_specs=[_ROWS, _WMAT, _ROWS, _ROWS],
    out_specs=[_ROWS, _COL1],
    out_shape=[
        jax.ShapeDtypeStruct((N, D), jnp.float32),
        jax.ShapeDtypeStruct((N, 1), jnp.float32),
    ],
)

_tc_mid = pl.pallas_call(
    _tc_mid_body,
    grid=(_GRID,),
    in_specs=[_ROWS, _ROWS, _ROWS, _COL1, _BVEC, _WMAT],
    out_specs=_ROWS,
    out_shape=jax.ShapeDtypeStruct((N, D), jnp.float32),
)

_tc_final = pl.pallas_call(
    _tc_final_body,
    grid=(_GRID,),
    in_specs=[_ROWS, _ROWS, _ROWS, _COL1, _BVEC],
    out_specs=_ROWS,
    out_shape=jax.ShapeDtypeStruct((N, D), jnp.float32),
)


def kernel(x, edge_index, W1, b1, W2, b2, W3, b3):
    src = edge_index[0]
    dst = edge_index[1]
    onesD = jnp.ones((N, D), jnp.float32)
    zerosD = jnp.zeros((N, D), jnp.float32)

    deg2 = _sc_aggregate(dst, dst, onesD, zerosD)
    hp, dinv = _tc_first(x, W1, deg2[0], deg2[1])

    S = _sc_aggregate(src, dst, hp, zerosD)
    hp = _tc_mid(S[0], S[1], hp, dinv, b1.reshape(1, D), W2)

    S = _sc_aggregate(src, dst, hp, zerosD)
    hp = _tc_mid(S[0], S[1], hp, dinv, b2.reshape(1, D), W3)

    S = _sc_aggregate(src, dst, hp, zerosD)
    return _tc_final(S[0], S[1], hp, dinv, b3.reshape(1, D))
